# baseline (speedup 1.0000x reference)
import functools

import jax
import jax.numpy as jnp
from jax import lax
from jax.experimental import pallas as pl
from jax.experimental.pallas import tpu as pltpu
from jax.experimental.pallas import tpu_sc as plsc

D_MODEL = 2048
D_SPACE = 64
N_PER = 256
N_EMB_USED = 1024
TOPK = 8
TILE = 1024

_ROUTER_EMB = (0, 0, 1, 2, 2, 3)

N_TOK = 16384
ROWS = 6 * N_TOK
NW = 32
RPW = ROWS // NW
CH = 8
NCH = RPW // CH
NVR = N_PER // 16


def _tc_body(x_ref, w_ref, b_ref, emb_ref, e_ref, embn_ref):
    @pl.when(pl.program_id(0) == 0)
    def _():
        emb = emb_ref[...]
        nrm = jnp.sqrt(jnp.sum(emb * emb, axis=1, keepdims=True))
        embn_ref[...] = emb / jnp.maximum(nrm, 1e-12)

    proj = jnp.dot(x_ref[...], w_ref[...],
                   preferred_element_type=jnp.float32) + b_ref[...]
    embn = embn_ref[...]
    for r in range(6):
        h = proj[:, r * D_SPACE:(r + 1) * D_SPACE]
        eb = _ROUTER_EMB[r]
        er = embn[eb * N_PER:(eb + 1) * N_PER, :]
        logits = jax.lax.dot_general(
            h, er, (((1,), (1,)), ((), ())),
            preferred_element_type=jnp.float32)
        e_ref[r, :, :] = jnp.exp(logits)


def _sc_route_body(e_hbm, out_hbm, buf, obuf, sem):
    wid = lax.axis_index("s") * 2 + lax.axis_index("c")
    base = wid * RPW
    lanes = lax.iota(jnp.int32, 16)
    bfly = [lanes ^ (8 >> k) for k in range(4)]

    def _allmax(v):
        for idx in bfly:
            v = jnp.maximum(v, v.at[idx].get(mode="promise_in_bounds"))
        return v

    def _allsum(v):
        for idx in bfly:
            v = v + v.at[idx].get(mode="promise_in_bounds")
        return v

    def chunk(ci, carry):
        r0 = base + ci * CH
        pltpu.sync_copy(e_hbm.at[pl.ds(r0, CH)], buf)
        for i in range(CH):
            ev = [buf[i, pl.ds(16 * j, 16)] for j in range(NVR)]
            acc = ev[0]
            for j in range(1, NVR):
                acc = acc + ev[j]
            z = _allsum(acc)
            cur = list(ev)
            e8 = jnp.zeros((16,), jnp.float32)
            for _ in range(TOPK):
                t = cur[0]
                for j in range(1, NVR):
                    t = jnp.maximum(t, cur[j])
                mkv = _allmax(t)
                cur = [jnp.where(c == mkv, -1.0, c) for c in cur]
                e8 = e8 + mkv
            rv = 1.0 / (e8 + 1e-8 * z)
            for j in range(NVR):
                obuf[i, pl.ds(16 * j, 16)] = jnp.where(
                    cur[j] < 0, ev[j] * rv, 0.0)
        pltpu.sync_copy(obuf, out_hbm.at[pl.ds(r0, CH)])
        return carry

    lax.fori_loop(0, NCH, chunk, 0)


@jax.jit
def kernel(x, W_all, b_all, neuron_emb):
    b, s, d = x.shape
    n_tok = b * s
    x2 = x.reshape(n_tok, d)
    b2 = b_all.reshape(1, -1)
    emb = neuron_emb[:N_EMB_USED]

    e3 = pl.pallas_call(
        _tc_body,
        grid=(n_tok // TILE,),
        in_specs=[
            pl.BlockSpec((TILE, d), lambda i: (i, 0)),
            pl.BlockSpec((d, 6 * D_SPACE), lambda i: (0, 0)),
            pl.BlockSpec((1, 6 * D_SPACE), lambda i: (0, 0)),
            pl.BlockSpec((N_EMB_USED, D_SPACE), lambda i: (0, 0)),
        ],
        out_specs=pl.BlockSpec((6, TILE, N_PER), lambda i: (0, i, 0)),
        out_shape=jax.ShapeDtypeStruct((6, n_tok, N_PER), jnp.float32),
        scratch_shapes=[pltpu.VMEM((N_EMB_USED, D_SPACE), jnp.float32)],
        compiler_params=pltpu.CompilerParams(
            dimension_semantics=("arbitrary",)),
    )(x2, W_all, b2, emb)

    e_flat = e3.reshape(ROWS, N_PER)

    mesh = plsc.VectorSubcoreMesh(core_axis_name="c", subcore_axis_name="s")
    sc_route = functools.partial(
        pl.kernel,
        mesh=mesh,
        out_type=jax.ShapeDtypeStruct((ROWS, N_PER), jnp.float32),
        scratch_types=[
            pltpu.VMEM((CH, N_PER), jnp.float32),
            pltpu.VMEM((CH, N_PER), jnp.float32),
            pltpu.SemaphoreType.DMA,
        ],
    )(_sc_route_body)

    out = sc_route(e_flat)
    out6 = out.reshape(6, b, s, N_PER)
    return tuple(out6[r] for r in range(6))


# speedup vs baseline: 7.1238x; 7.1238x over previous
import functools

import jax
import jax.numpy as jnp
from jax.experimental import pallas as pl
from jax.experimental.pallas import tpu as pltpu

D_MODEL = 2048
D_SPACE = 64
N_PER = 256
N_EMB_USED = 1024
TOPK = 8
TILE = 1024

_ROUTER_EMB = (0, 0, 1, 2, 2, 3)


def _body(x_ref, w_ref, b_ref, emb_ref, *refs):
    out_refs, embn_ref = refs[:6], refs[6]
    t = x_ref.shape[0]

    @pl.when(pl.program_id(0) == 0)
    def _():
        emb = emb_ref[...]
        nrm = jnp.sqrt(jnp.sum(emb * emb, axis=1, keepdims=True))
        embn_ref[...] = emb / jnp.maximum(nrm, 1e-12)

    proj = jnp.dot(x_ref[...], w_ref[...],
                   preferred_element_type=jnp.float32) + b_ref[...]
    embn = embn_ref[...]

    for r in range(6):
        h = proj[:, r * D_SPACE:(r + 1) * D_SPACE]
        eb = _ROUTER_EMB[r]
        er = embn[eb * N_PER:(eb + 1) * N_PER, :]
        logits = jax.lax.dot_general(
            h, er, (((1,), (1,)), ((), ())),
            preferred_element_type=jnp.float32)

        e = jnp.exp(logits)
        z = jnp.sum(e, axis=1, keepdims=True)

        cur = e
        e8 = jnp.zeros((t, 1), dtype=jnp.float32)
        for _ in range(TOPK):
            mk = jnp.max(cur, axis=1, keepdims=True)
            cur = jnp.where(cur == mk, -1.0, cur)
            e8 = e8 + mk

        recip = 1.0 / (e8 + 1e-8 * z)
        out_refs[r][...] = (e - jnp.maximum(cur, 0.0)) * recip


@jax.jit
def kernel(x, W_all, b_all, neuron_emb):
    b, s, d = x.shape
    n_tok = b * s
    x2 = x.reshape(n_tok, d)
    b2 = b_all.reshape(1, -1)
    emb = neuron_emb[:N_EMB_USED]

    grid = (n_tok // TILE,)
    out_sds = [jax.ShapeDtypeStruct((n_tok, N_PER), jnp.float32)
               for _ in range(6)]
    outs = pl.pallas_call(
        _body,
        grid=grid,
        in_specs=[
            pl.BlockSpec((TILE, d), lambda i: (i, 0)),
            pl.BlockSpec((d, 6 * D_SPACE), lambda i: (0, 0)),
            pl.BlockSpec((1, 6 * D_SPACE), lambda i: (0, 0)),
            pl.BlockSpec((N_EMB_USED, D_SPACE), lambda i: (0, 0)),
        ],
        out_specs=[pl.BlockSpec((TILE, N_PER), lambda i: (i, 0))
                   for _ in range(6)],
        out_shape=out_sds,
        scratch_shapes=[pltpu.VMEM((N_EMB_USED, D_SPACE), jnp.float32)],
        compiler_params=pltpu.CompilerParams(
            dimension_semantics=("parallel",)),
    )(x2, W_all, b2, emb)
    return tuple(o.reshape(b, s, N_PER) for o in outs)
